# baseline (speedup 1.0000x reference)
import functools

import jax
import jax.numpy as jnp
from jax.experimental import pallas as pl


def _bf16(v):
    return v.astype(jnp.bfloat16)


def _xw_kernel(x_ref, w_ref, o_ref):
    o_ref[...] = jnp.dot(_bf16(x_ref[...]), _bf16(w_ref[...]),
                         preferred_element_type=jnp.float32)


def _mid_layer_kernel(adj_ref, s_ref, b_ref, w_ref, o_ref):
    acc = jnp.dot(_bf16(adj_ref[...]), _bf16(s_ref[...]),
                  preferred_element_type=jnp.float32)
    h = jnp.maximum(acc + b_ref[...], 0.0)
    o_ref[...] = jnp.dot(_bf16(h), _bf16(w_ref[...]),
                         preferred_element_type=jnp.float32)


def _last_layer_kernel(adj_ref, s_ref, b_ref, o_ref):
    acc = jnp.dot(_bf16(adj_ref[...]), _bf16(s_ref[...]),
                  preferred_element_type=jnp.float32)
    h = acc + b_ref[...]
    m = jnp.max(h, axis=1, keepdims=True)
    lse = jnp.log(jnp.sum(jnp.exp(h - m), axis=1, keepdims=True)) + m
    o_ref[...] = h - lse


def _row_block(n):
    for bm in (400, 200, 80, 40, 8):
        if n % bm == 0:
            return bm
    return n


def _layer_call(body, adj, s, b, extra, out_cols):
    n = adj.shape[0]
    bm = _row_block(n)
    grid = (n // bm,)
    k = s.shape[1]
    in_specs = [
        pl.BlockSpec((bm, n), lambda i: (i, 0)),
        pl.BlockSpec((n, k), lambda i: (0, 0)),
        pl.BlockSpec((1, b.shape[1]), lambda i: (0, 0)),
    ]
    args = [adj, s, b]
    if extra is not None:
        in_specs.append(pl.BlockSpec(extra.shape, lambda i: (0, 0)))
        args.append(extra)
    return pl.pallas_call(
        body,
        grid=grid,
        in_specs=in_specs,
        out_specs=pl.BlockSpec((bm, out_cols), lambda i: (i, 0)),
        out_shape=jax.ShapeDtypeStruct((n, out_cols), jnp.float32),
    )(*args)


@jax.jit
def kernel(x, adj, W1, b1, W2, b2, W3, b3):
    n = adj.shape[0]
    s1 = pl.pallas_call(
        _xw_kernel,
        out_shape=jax.ShapeDtypeStruct((n, W1.shape[1]), jnp.float32),
    )(x, W1)
    b1r = b1.reshape(1, -1)
    b2r = b2.reshape(1, -1)
    b3r = b3.reshape(1, -1)
    s2 = _layer_call(_mid_layer_kernel, adj, s1, b1r, W2, W2.shape[1])
    s3 = _layer_call(_mid_layer_kernel, adj, s2, b2r, W3, W3.shape[1])
    out = _layer_call(_last_layer_kernel, adj, s3, b3r, None, W3.shape[1])
    return out


# speedup vs baseline: 1.0724x; 1.0724x over previous
import functools

import jax
import jax.numpy as jnp
from jax.experimental import pallas as pl


def _bf16(v):
    return v.astype(jnp.bfloat16)


def _xw_kernel(x_ref, w_ref, o_ref):
    o_ref[...] = jnp.dot(_bf16(x_ref[...]), _bf16(w_ref[...]),
                         preferred_element_type=jnp.float32)


def _first_layer_kernel(adj_ref, s_ref, b_ref, w_ref, o_ref, adjb_ref):
    adjb = _bf16(adj_ref[...])
    adjb_ref[...] = adjb
    acc = jnp.dot(adjb, _bf16(s_ref[...]), preferred_element_type=jnp.float32)
    h = jnp.maximum(acc + b_ref[...], 0.0)
    o_ref[...] = jnp.dot(_bf16(h), _bf16(w_ref[...]),
                         preferred_element_type=jnp.float32)


def _mid_layer_kernel(adj_ref, s_ref, b_ref, w_ref, o_ref):
    acc = jnp.dot(adj_ref[...], _bf16(s_ref[...]),
                  preferred_element_type=jnp.float32)
    h = jnp.maximum(acc + b_ref[...], 0.0)
    o_ref[...] = jnp.dot(_bf16(h), _bf16(w_ref[...]),
                         preferred_element_type=jnp.float32)


def _last_layer_kernel(adj_ref, s_ref, b_ref, o_ref):
    acc = jnp.dot(adj_ref[...], _bf16(s_ref[...]),
                  preferred_element_type=jnp.float32)
    h = acc + b_ref[...]
    m = jnp.max(h, axis=1, keepdims=True)
    lse = jnp.log(jnp.sum(jnp.exp(h - m), axis=1, keepdims=True)) + m
    o_ref[...] = h - lse


def _row_block(n):
    for bm in (400, 200, 80, 40, 8):
        if n % bm == 0:
            return bm
    return n


def _layer_call(body, adj, s, b, extra, out_cols, emit_adj_bf16=False):
    n = adj.shape[0]
    bm = _row_block(n)
    grid = (n // bm,)
    k = s.shape[1]
    in_specs = [
        pl.BlockSpec((bm, n), lambda i: (i, 0)),
        pl.BlockSpec((n, k), lambda i: (0, 0)),
        pl.BlockSpec((1, b.shape[1]), lambda i: (0, 0)),
    ]
    args = [adj, s, b]
    if extra is not None:
        in_specs.append(pl.BlockSpec(extra.shape, lambda i: (0, 0)))
        args.append(extra)
    out_specs = pl.BlockSpec((bm, out_cols), lambda i: (i, 0))
    out_shape = jax.ShapeDtypeStruct((n, out_cols), jnp.float32)
    if emit_adj_bf16:
        out_specs = [out_specs, pl.BlockSpec((bm, n), lambda i: (i, 0))]
        out_shape = [out_shape, jax.ShapeDtypeStruct((n, n), jnp.bfloat16)]
    return pl.pallas_call(
        body,
        grid=grid,
        in_specs=in_specs,
        out_specs=out_specs,
        out_shape=out_shape,
    )(*args)


@jax.jit
def kernel(x, adj, W1, b1, W2, b2, W3, b3):
    n = adj.shape[0]
    s1 = pl.pallas_call(
        _xw_kernel,
        out_shape=jax.ShapeDtypeStruct((n, W1.shape[1]), jnp.float32),
    )(x, W1)
    b1r = b1.reshape(1, -1)
    b2r = b2.reshape(1, -1)
    b3r = b3.reshape(1, -1)
    s2, adj_bf = _layer_call(_first_layer_kernel, adj, s1, b1r, W2,
                             W2.shape[1], emit_adj_bf16=True)
    s3 = _layer_call(_mid_layer_kernel, adj_bf, s2, b2r, W3, W3.shape[1])
    out = _layer_call(_last_layer_kernel, adj_bf, s3, b3r, None, W3.shape[1])
    return out


# speedup vs baseline: 1.1074x; 1.0326x over previous
import jax
import jax.numpy as jnp
from jax.experimental import pallas as pl


def _bf16(v):
    return v.astype(jnp.bfloat16)


def _xw_kernel(x_ref, w_ref, o_ref):
    o_ref[...] = _bf16(jnp.dot(_bf16(x_ref[...]), w_ref[...],
                               preferred_element_type=jnp.float32))


def _first_layer_kernel(adj_ref, s_ref, b_ref, w_ref, o_ref, adjb_ref):
    adjb = _bf16(adj_ref[...])
    adjb_ref[...] = adjb
    acc = jnp.dot(adjb, s_ref[...], preferred_element_type=jnp.float32)
    h = _bf16(jnp.maximum(acc + b_ref[...], 0.0))
    o_ref[...] = _bf16(jnp.dot(h, w_ref[...],
                               preferred_element_type=jnp.float32))


def _mid_layer_kernel(adj_ref, s_ref, b_ref, w_ref, o_ref):
    acc = jnp.dot(adj_ref[...], s_ref[...], preferred_element_type=jnp.float32)
    h = _bf16(jnp.maximum(acc + b_ref[...], 0.0))
    o_ref[...] = _bf16(jnp.dot(h, w_ref[...],
                               preferred_element_type=jnp.float32))


def _last_layer_kernel(adj_ref, s_ref, b_ref, o_ref):
    acc = jnp.dot(adj_ref[...], s_ref[...], preferred_element_type=jnp.float32)
    h = acc + b_ref[...]
    m = jnp.max(h, axis=1, keepdims=True)
    lse = jnp.log(jnp.sum(jnp.exp(h - m), axis=1, keepdims=True)) + m
    o_ref[...] = h - lse


def _row_block(n, target):
    for bm in (target, 400, 200, 80, 40, 8):
        if bm <= target and n % bm == 0:
            return bm
    return n


def _layer_call(body, adj, s, b, extra, out_cols, out_dtype, bm_target,
                emit_adj_bf16=False):
    n = adj.shape[0]
    bm = _row_block(n, bm_target)
    grid = (n // bm,)
    k = s.shape[1]
    in_specs = [
        pl.BlockSpec((bm, n), lambda i: (i, 0)),
        pl.BlockSpec((n, k), lambda i: (0, 0)),
        pl.BlockSpec((1, b.shape[1]), lambda i: (0, 0)),
    ]
    args = [adj, s, b]
    if extra is not None:
        in_specs.append(pl.BlockSpec(extra.shape, lambda i: (0, 0)))
        args.append(extra)
    out_specs = pl.BlockSpec((bm, out_cols), lambda i: (i, 0))
    out_shape = jax.ShapeDtypeStruct((n, out_cols), out_dtype)
    if emit_adj_bf16:
        out_specs = [out_specs, pl.BlockSpec((bm, n), lambda i: (i, 0))]
        out_shape = [out_shape, jax.ShapeDtypeStruct((n, n), jnp.bfloat16)]
    return pl.pallas_call(
        body,
        grid=grid,
        in_specs=in_specs,
        out_specs=out_specs,
        out_shape=out_shape,
    )(*args)


@jax.jit
def kernel(x, adj, W1, b1, W2, b2, W3, b3):
    n = adj.shape[0]
    w1, w2, w3 = _bf16(W1), _bf16(W2), _bf16(W3)
    s1 = pl.pallas_call(
        _xw_kernel,
        out_shape=jax.ShapeDtypeStruct((n, W1.shape[1]), jnp.bfloat16),
    )(x, w1)
    b1r = b1.reshape(1, -1)
    b2r = b2.reshape(1, -1)
    b3r = b3.reshape(1, -1)
    s2, adj_bf = _layer_call(_first_layer_kernel, adj, s1, b1r, w2,
                             W2.shape[1], jnp.bfloat16, 200,
                             emit_adj_bf16=True)
    s3 = _layer_call(_mid_layer_kernel, adj_bf, s2, b2r, w3,
                     W3.shape[1], jnp.bfloat16, 400)
    out = _layer_call(_last_layer_kernel, adj_bf, s3, b3r, None,
                      W3.shape[1], jnp.float32, 400)
    return out
